# baseline (speedup 1.0000x reference)
import jax
import jax.numpy as jnp
from jax.experimental import pallas as pl

_FWD_WEIGHT = 1.0


def _l1_kernel(x_ref, g_ref, m_ref, o_ref):
    i = pl.program_id(0)
    d = jnp.abs(x_ref[...] - g_ref[...])
    mf = m_ref[...].astype(jnp.float32)
    r, l = d.shape
    idx = jax.lax.broadcasted_iota(jnp.int32, (r, l), 1) // 2
    mex = jnp.take_along_axis(mf, idx, axis=1)
    s = jnp.sum(d * mex).reshape(1, 1)

    @pl.when(i == 0)
    def _init():
        o_ref[...] = jnp.zeros((1, 1), jnp.float32)

    o_ref[...] += s


def kernel(uv_points, uv_gt, object_mask):
    B, H, W, C = uv_points.shape
    n = B * H * W * C
    rows = n // 128
    x = uv_points.reshape(rows, 128)
    g = uv_gt.reshape(rows, 128)
    m = object_mask.view(jnp.uint8).reshape(rows, 64)

    R = 4096
    grid = (rows // R,)
    out = pl.pallas_call(
        _l1_kernel,
        grid=grid,
        in_specs=[
            pl.BlockSpec((R, 128), lambda i: (i, 0)),
            pl.BlockSpec((R, 128), lambda i: (i, 0)),
            pl.BlockSpec((R, 64), lambda i: (i, 0)),
        ],
        out_specs=pl.BlockSpec((1, 1), lambda i: (0, 0)),
        out_shape=jax.ShapeDtypeStruct((1, 1), jnp.float32),
    )(x, g, m)

    uv_loss = out[0, 0] / float(B * H)
    return (_FWD_WEIGHT * uv_loss, uv_loss)


# speedup vs baseline: 84.3227x; 84.3227x over previous
import jax
import jax.numpy as jnp
from jax.experimental import pallas as pl

_FWD_WEIGHT = 1.0


def _l1_kernel(x_ref, g_ref, m_ref, o_ref):
    i = pl.program_id(0)
    d = jnp.abs(x_ref[...] - g_ref[...])
    mf = m_ref[...].astype(jnp.float32)
    s = jnp.sum(d * mf).reshape(1, 1)

    @pl.when(i == 0)
    def _init():
        o_ref[...] = jnp.zeros((1, 1), jnp.float32)

    o_ref[...] += s


def kernel(uv_points, uv_gt, object_mask):
    B, H, W, C = uv_points.shape
    nrow = B * H * (W // 128) * C
    mrow = B * H * (W // 128)

    def as_rows(a):
        return (
            a.reshape(B, H, W // 128, 128, C)
            .transpose(0, 1, 2, 4, 3)
            .reshape(nrow, 128)
        )

    x = as_rows(uv_points)
    g = as_rows(uv_gt)
    m = jnp.broadcast_to(
        object_mask.view(jnp.uint8).reshape(mrow, 1, 128), (mrow, 2, 128)
    ).reshape(nrow, 128)

    R = 8192
    grid = (nrow // R,)
    out = pl.pallas_call(
        _l1_kernel,
        grid=grid,
        in_specs=[
            pl.BlockSpec((R, 128), lambda i: (i, 0)),
            pl.BlockSpec((R, 128), lambda i: (i, 0)),
            pl.BlockSpec((R, 128), lambda i: (i, 0)),
        ],
        out_specs=pl.BlockSpec((1, 1), lambda i: (0, 0)),
        out_shape=jax.ShapeDtypeStruct((1, 1), jnp.float32),
    )(x, g, m)

    uv_loss = out[0, 0] / float(B * H)
    return (_FWD_WEIGHT * uv_loss, uv_loss)


# speedup vs baseline: 268.0766x; 3.1792x over previous
import jax
import jax.numpy as jnp
from jax.experimental import pallas as pl

_FWD_WEIGHT = 1.0


def _l1_kernel(x_ref, g_ref, m_ref, o_ref):
    i = pl.program_id(0)
    d = jnp.abs(x_ref[...] - g_ref[...])
    mf = m_ref[...].astype(jnp.float32)
    r = mf.shape[0] // 4
    m3 = mf.reshape(r, 4, 128)
    idx = jax.lax.broadcasted_iota(jnp.int32, (r, 8, 128), 1) // 2
    mex = jnp.take_along_axis(m3, idx, axis=1).reshape(r * 8, 128)
    s = jnp.sum(d * mex).reshape(1, 1)

    @pl.when(i == 0)
    def _init():
        o_ref[...] = jnp.zeros((1, 1), jnp.float32)

    o_ref[...] += s


def kernel(uv_points, uv_gt, object_mask):
    B, H, W, C = uv_points.shape
    nrow = B * H * (W // 128) * C
    mrow = B * H * (W // 128)

    def as_rows(a):
        return (
            a.reshape(B, H, W // 128, 128, C)
            .transpose(0, 1, 2, 4, 3)
            .reshape(nrow, 128)
        )

    x = as_rows(uv_points)
    g = as_rows(uv_gt)
    m = object_mask.view(jnp.uint8).reshape(mrow, 128)

    R = 8192
    grid = (nrow // R,)
    out = pl.pallas_call(
        _l1_kernel,
        grid=grid,
        in_specs=[
            pl.BlockSpec((R, 128), lambda i: (i, 0)),
            pl.BlockSpec((R, 128), lambda i: (i, 0)),
            pl.BlockSpec((R // 2, 128), lambda i: (i, 0)),
        ],
        out_specs=pl.BlockSpec((1, 1), lambda i: (0, 0)),
        out_shape=jax.ShapeDtypeStruct((1, 1), jnp.float32),
    )(x, g, m)

    uv_loss = out[0, 0] / float(B * H)
    return (_FWD_WEIGHT * uv_loss, uv_loss)
